# baseline (speedup 1.0000x reference)
import functools

import jax
import jax.numpy as jnp
from jax import lax
from jax.experimental import pallas as pl
from jax.experimental.pallas import tpu as pltpu
from jax.experimental.pallas import tpu_sc as plsc


@functools.lru_cache(maxsize=None)
def _sc_agg(n, d, e_total):
    info = plsc.get_sparse_core_info()
    nc, ns, lanes = info.num_cores, info.num_subcores, info.num_lanes
    nw = nc * ns
    assert e_total % nw == 0
    epw = e_total // nw
    ch = 64
    nfull = epw // ch
    tail = epw - nfull * ch
    assert 0 < tail <= lanes and tail % 8 == 0
    nbuf = 2
    assert nfull % nbuf == 0
    nouter = nfull // nbuf
    stride = (n // ns) // 8 * 8
    span = stride + (n - ns * stride)
    assert stride % 8 == 0 and span >= stride and (ns - 1) * stride + span == n
    nvr = d // lanes
    slab = nfull * ch

    mesh = plsc.VectorSubcoreMesh(core_axis_name="c", subcore_axis_name="s")

    @functools.partial(
        pl.kernel,
        mesh=mesh,
        out_type=jax.ShapeDtypeStruct((nc * n, d), jnp.float32),
        scratch_types=[
            pltpu.VMEM((slab,), jnp.int32),
            pltpu.VMEM((slab,), jnp.int32),
            pltpu.VMEM((slab,), jnp.float32),
            [pltpu.VMEM((ch,), jnp.int32) for _ in range(nbuf)],
            [pltpu.VMEM((ch,), jnp.int32) for _ in range(nbuf)],
            [pltpu.VMEM((ch, d), jnp.float32) for _ in range(nbuf)],
            pltpu.VMEM((tail,), jnp.int32),
            pltpu.VMEM((tail,), jnp.int32),
            pltpu.VMEM((lanes,), jnp.float32),
            pltpu.VMEM((tail, d), jnp.float32),
            pltpu.VMEM_SHARED((n, d), jnp.float32),
            [pltpu.SemaphoreType.DMA for _ in range(nbuf)],
            pltpu.SemaphoreType.DMA,
        ],
    )
    def body(feat_hbm, src_hbm, dst_hbm, e_hbm, zeros_hbm, out_hbm,
             src_all, dst_all, e_all,
             src_b, dst_b, rows_g,
             src_t, dst_t, e_t, rows_t,
             agg_sh, gsem, sem):
        ci = lax.axis_index("c")
        si = lax.axis_index("s")
        wid = si * nc + ci

        r0 = si * stride
        pltpu.sync_copy(zeros_hbm, agg_sh.at[pl.ds(r0, span)])

        base = wid * epw
        pltpu.sync_copy(src_hbm.at[pl.ds(base, slab)], src_all)
        pltpu.sync_copy(dst_hbm.at[pl.ds(base, slab)], dst_all)
        pltpu.sync_copy(e_hbm.at[pl.ds(base, slab)], e_all)
        plsc.subcore_barrier()

        def scale_rows(dst_rows, srcr, evals, ebase, count):
            def sgrp(g, _):
                ev16 = evals[pl.ds(ebase + g * lanes, lanes)]
                for l in range(min(lanes, count)):
                    es = ev16[l]
                    r = g * lanes + l
                    for v in range(nvr):
                        sl = pl.ds(v * lanes, lanes)
                        dst_rows[r, sl] = srcr[r, sl] * es
                return 0
            lax.fori_loop(0, max(count // lanes, 1), sgrp, 0)

        def vcopy(src_ref, soff, dst_ref):
            for v in range(ch // lanes):
                dst_ref[pl.ds(v * lanes, lanes)] = (
                    src_ref[pl.ds(soff + v * lanes, lanes)])

        def drain(sem_, buf):
            pltpu.make_async_copy(feat_hbm.at[pl.ds(0, ch)], buf, sem_).wait()

        for b in range(nbuf):
            vcopy(src_all, b * ch, src_b[b])
            pltpu.async_copy(feat_hbm.at[src_b[b]], rows_g[b], gsem[b])

        def outer(i, _):
            for b in range(nbuf):
                c = i * nbuf + b
                drain(gsem[b], rows_g[b])
                scale_rows(rows_g[b], rows_g[b], e_all, c * ch, ch)
                vcopy(dst_all, c * ch, dst_b[b])
                pltpu.sync_copy(rows_g[b], agg_sh.at[dst_b[b]], add=True)
                @pl.when(i < nouter - 1)
                def _():
                    vcopy(src_all, (c + nbuf) * ch, src_b[b])
                    pltpu.async_copy(feat_hbm.at[src_b[b]], rows_g[b], gsem[b])
            return 0
        lax.fori_loop(0, nouter, outer, 0)

        if tail:
            off = base + nfull * ch
            pltpu.sync_copy(src_hbm.at[pl.ds(off, tail)], src_t)
            pltpu.sync_copy(dst_hbm.at[pl.ds(off, tail)], dst_t)
            pltpu.sync_copy(e_hbm.at[pl.ds(off, tail)],
                            e_t.at[pl.ds(0, tail)])
            pltpu.async_copy(feat_hbm.at[src_t], rows_t, sem).wait()
            scale_rows(rows_t, rows_t, e_t, 0, tail)
            pltpu.sync_copy(rows_t, agg_sh.at[dst_t], add=True)

        plsc.subcore_barrier()

        done = 0
        while done < span:
            sz = min(128, span - done)
            pltpu.sync_copy(agg_sh.at[pl.ds(r0 + done, sz)],
                            out_hbm.at[pl.ds(ci * n + r0 + done, sz)])
            done += sz

    return body


@functools.lru_cache(maxsize=None)
def _tc_k(n, d, bm):
    nb = n // bm

    def body(x_ref, feat_ref, out_ref):
        out_ref[:, d:] = lax.dot_general(
            x_ref[...], feat_ref[...],
            (((1,), (1,)), ((), ())),
            preferred_element_type=jnp.float32,
            precision=lax.Precision.DEFAULT,
        )

    return pl.pallas_call(
        body,
        grid=(nb,),
        in_specs=[
            pl.BlockSpec((bm, d), lambda i: (i, 0)),
            pl.BlockSpec((n, d), lambda i: (0, 0)),
        ],
        out_specs=pl.BlockSpec((bm, n + d), lambda i: (i, 0)),
        out_shape=jax.ShapeDtypeStruct((n, n + d), jnp.float32),
        compiler_params=pltpu.CompilerParams(
            dimension_semantics=("arbitrary",),
        ),
    )


@functools.lru_cache(maxsize=None)
def _tc_fix(n, d):
    def body(buf_ref, p0_ref, p1_ref, out_ref):
        out_ref[...] = p0_ref[...] + p1_ref[...]

    return pl.pallas_call(
        body,
        grid=(1,),
        in_specs=[
            pl.BlockSpec(memory_space=pl.ANY),
            pl.BlockSpec((n, d), lambda i: (0, 0)),
            pl.BlockSpec((n, d), lambda i: (1, 0)),
        ],
        out_specs=pl.BlockSpec((n, d), lambda i: (0, 0)),
        out_shape=jax.ShapeDtypeStruct((n, n + d), jnp.float32),
        input_output_aliases={0: 0},
        compiler_params=pltpu.CompilerParams(
            dimension_semantics=("arbitrary",),
        ),
    )


def kernel(t, x, edge_index, e):
    n, w = x.shape
    d = w - n
    e_total = edge_index.shape[1]
    feat = x[:, :d]
    src = edge_index[0]
    dst = edge_index[1]
    ev = e[:, 0]
    zeros = jnp.zeros((640, d), jnp.float32)
    partials = _sc_agg(n, d, e_total)(feat, src, dst, ev, zeros)
    buf = _tc_k(n, d, 200)(x, feat)
    return _tc_fix(n, d)(buf, partials, partials)


# speedup vs baseline: 1.0046x; 1.0046x over previous
import functools

import jax
import jax.numpy as jnp
from jax import lax
from jax.experimental import pallas as pl
from jax.experimental.pallas import tpu as pltpu
from jax.experimental.pallas import tpu_sc as plsc


@functools.lru_cache(maxsize=None)
def _sc_agg(n, d, e_total):
    info = plsc.get_sparse_core_info()
    nc, ns, lanes = info.num_cores, info.num_subcores, info.num_lanes
    nw = nc * ns
    assert e_total % nw == 0
    epw = e_total // nw
    ch = 64
    nfull = epw // ch
    tail = epw - nfull * ch
    assert 0 < tail <= lanes and tail % 8 == 0
    nbuf = 2
    assert nfull % nbuf == 0
    nouter = nfull // nbuf
    stride = (n // ns) // 8 * 8
    span = stride + (n - ns * stride)
    assert stride % 8 == 0 and span >= stride and (ns - 1) * stride + span == n
    nvr = d // lanes
    slab = nfull * ch

    mesh = plsc.VectorSubcoreMesh(core_axis_name="c", subcore_axis_name="s")

    @functools.partial(
        pl.kernel,
        mesh=mesh,
        out_type=jax.ShapeDtypeStruct((nc * n, d), jnp.float32),
        scratch_types=[
            pltpu.VMEM((slab,), jnp.int32),
            pltpu.VMEM((slab,), jnp.int32),
            pltpu.VMEM((slab,), jnp.float32),
            [pltpu.VMEM((ch,), jnp.int32) for _ in range(nbuf)],
            [pltpu.VMEM((ch,), jnp.int32) for _ in range(nbuf)],
            [pltpu.VMEM((ch, d), jnp.float32) for _ in range(nbuf)],
            pltpu.VMEM((tail,), jnp.int32),
            pltpu.VMEM((tail,), jnp.int32),
            pltpu.VMEM((lanes,), jnp.float32),
            pltpu.VMEM((tail, d), jnp.float32),
            pltpu.VMEM_SHARED((n, d), jnp.float32),
            [pltpu.SemaphoreType.DMA for _ in range(nbuf)],
            pltpu.SemaphoreType.DMA,
        ],
    )
    def body(feat_hbm, src_hbm, dst_hbm, e_hbm, zeros_hbm, out_hbm,
             src_all, dst_all, e_all,
             src_b, dst_b, rows_g,
             src_t, dst_t, e_t, rows_t,
             agg_sh, gsem, sem):
        ci = lax.axis_index("c")
        si = lax.axis_index("s")
        wid = si * nc + ci

        r0 = si * stride
        pltpu.sync_copy(zeros_hbm, agg_sh.at[pl.ds(r0, span)])

        base = wid * epw
        pltpu.sync_copy(src_hbm.at[pl.ds(base, slab)], src_all)
        pltpu.sync_copy(dst_hbm.at[pl.ds(base, slab)], dst_all)
        pltpu.sync_copy(e_hbm.at[pl.ds(base, slab)], e_all)
        plsc.subcore_barrier()

        def scale_rows(dst_rows, srcr, evals, ebase, count):
            def sgrp(g, _):
                ev16 = evals[pl.ds(ebase + g * lanes, lanes)]
                for l in range(min(lanes, count)):
                    es = ev16[l]
                    r = g * lanes + l
                    for v in range(nvr):
                        sl = pl.ds(v * lanes, lanes)
                        dst_rows[r, sl] = srcr[r, sl] * es
                return 0
            lax.fori_loop(0, max(count // lanes, 1), sgrp, 0)

        def vcopy(src_ref, soff, dst_ref):
            for v in range(ch // lanes):
                dst_ref[pl.ds(v * lanes, lanes)] = (
                    src_ref[pl.ds(soff + v * lanes, lanes)])

        def drain(sem_, buf):
            pltpu.make_async_copy(feat_hbm.at[pl.ds(0, ch)], buf, sem_).wait()

        for b in range(nbuf):
            vcopy(src_all, b * ch, src_b[b])
            pltpu.async_copy(feat_hbm.at[src_b[b]], rows_g[b], gsem[b])

        def outer(i, _):
            for b in range(nbuf):
                c = i * nbuf + b
                drain(gsem[b], rows_g[b])
                scale_rows(rows_g[b], rows_g[b], e_all, c * ch, ch)
                vcopy(dst_all, c * ch, dst_b[b])
                pltpu.sync_copy(rows_g[b], agg_sh.at[dst_b[b]], add=True)
                @pl.when(i < nouter - 1)
                def _():
                    vcopy(src_all, (c + nbuf) * ch, src_b[b])
                    pltpu.async_copy(feat_hbm.at[src_b[b]], rows_g[b], gsem[b])
            return 0
        lax.fori_loop(0, nouter, outer, 0)

        if tail:
            off = base + nfull * ch
            pltpu.sync_copy(src_hbm.at[pl.ds(off, tail)], src_t)
            pltpu.sync_copy(dst_hbm.at[pl.ds(off, tail)], dst_t)
            pltpu.sync_copy(e_hbm.at[pl.ds(off, tail)],
                            e_t.at[pl.ds(0, tail)])
            pltpu.async_copy(feat_hbm.at[src_t], rows_t, sem).wait()
            scale_rows(rows_t, rows_t, e_t, 0, tail)
            pltpu.sync_copy(rows_t, agg_sh.at[dst_t], add=True)

        plsc.subcore_barrier()

        done = 0
        while done < span:
            sz = min(128, span - done)
            pltpu.sync_copy(agg_sh.at[pl.ds(r0 + done, sz)],
                            out_hbm.at[pl.ds(ci * n + r0 + done, sz)])
            done += sz

    return body


@functools.lru_cache(maxsize=None)
def _tc_k(n, d, bm):
    nb = n // bm

    def body(x_ref, feat_ref, out_ref):
        out_ref[:, d:] = lax.dot_general(
            x_ref[...], feat_ref[...],
            (((1,), (1,)), ((), ())),
            preferred_element_type=jnp.float32,
            precision=lax.Precision.DEFAULT,
        )

    return pl.pallas_call(
        body,
        grid=(nb,),
        in_specs=[
            pl.BlockSpec((bm, d), lambda i: (i, 0)),
            pl.BlockSpec((n, d), lambda i: (0, 0)),
        ],
        out_specs=pl.BlockSpec((bm, n + d), lambda i: (i, 0)),
        out_shape=jax.ShapeDtypeStruct((n, n + d), jnp.float32),
        compiler_params=pltpu.CompilerParams(
            dimension_semantics=("arbitrary",),
        ),
    )


@functools.lru_cache(maxsize=None)
def _tc_fix(n, d):
    def body(buf_ref, p0_ref, p1_ref, out_ref):
        out_ref[...] = p0_ref[...] + p1_ref[...]

    return pl.pallas_call(
        body,
        grid=(1,),
        in_specs=[
            pl.BlockSpec(memory_space=pl.ANY),
            pl.BlockSpec((n, d), lambda i: (0, 0)),
            pl.BlockSpec((n, d), lambda i: (1, 0)),
        ],
        out_specs=pl.BlockSpec((n, d), lambda i: (0, 0)),
        out_shape=jax.ShapeDtypeStruct((n, n + d), jnp.float32),
        input_output_aliases={0: 0},
        compiler_params=pltpu.CompilerParams(
            dimension_semantics=("arbitrary",),
        ),
    )


def kernel(t, x, edge_index, e):
    n, w = x.shape
    d = w - n
    e_total = edge_index.shape[1]
    feat = x[:, :d]
    src = edge_index[0]
    dst = edge_index[1]
    ev = e[:, 0]
    zeros = jnp.zeros((640, d), jnp.float32)
    partials = _sc_agg(n, d, e_total)(feat, src, dst, ev, zeros)
    buf = _tc_k(n, d, 400)(x, feat)
    return _tc_fix(n, d)(buf, partials, partials)
